# baseline (speedup 1.0000x reference)
import jax
import jax.numpy as jnp
from jax.experimental import pallas as pl
from jax.experimental.pallas import tpu as pltpu

WINDOW = 50
BATCH = 4096
DIM = 128
SLOTS = 2
NCHUNK = WINDOW // SLOTS
NBUF = 12


def _body(ptr_ref, x_ref, data_ref, out_ref, *scratch):
    bufs = scratch[:NBUF]
    rsems = scratch[NBUF:2 * NBUF]
    wsems = scratch[2 * NBUF:3 * NBUF]
    p = ptr_ref[0]

    def start_reads(c, buf, sem):
        for s in range(SLOTS):
            slot = c * SLOTS + s

            @pl.when(slot == p)
            def _from_x():
                pltpu.make_async_copy(
                    x_ref, buf.at[pl.ds(s * BATCH, BATCH)], sem).start()

            @pl.when(slot != p)
            def _from_data():
                pltpu.make_async_copy(
                    data_ref.at[pl.ds(slot * BATCH, BATCH)],
                    buf.at[pl.ds(s * BATCH, BATCH)], sem).start()

    def wait_reads(buf, sem):
        for s in range(SLOTS):
            pltpu.make_async_copy(
                data_ref.at[pl.ds(0, BATCH)],
                buf.at[pl.ds(s * BATCH, BATCH)], sem).wait()

    def wait_write(buf, sem):
        pltpu.make_async_copy(buf, out_ref.at[pl.ds(0, SLOTS * BATCH)],
                              sem).wait()

    for c in range(NBUF - 1):
        start_reads(c, bufs[c % NBUF], rsems[c % NBUF])
    for c in range(NCHUNK):
        b = c % NBUF
        wait_reads(bufs[b], rsems[b])
        pltpu.make_async_copy(
            bufs[b], out_ref.at[pl.ds(c * SLOTS * BATCH, SLOTS * BATCH)],
            wsems[b]).start()
        nxt = c + NBUF - 1
        if nxt < NCHUNK:
            bn = nxt % NBUF
            if nxt >= NBUF:
                wait_write(bufs[bn], wsems[bn])
            start_reads(nxt, bufs[bn], rsems[bn])
    for c in range(max(0, NCHUNK - NBUF), NCHUNK):
        b = c % NBUF
        wait_write(bufs[b], wsems[b])


def kernel(x, data, pointer):
    ptr = jnp.atleast_1d(jnp.asarray(pointer, dtype=jnp.int32))
    flat = data.reshape(WINDOW * BATCH, DIM)
    grid_spec = pltpu.PrefetchScalarGridSpec(
        num_scalar_prefetch=1,
        grid=(1,),
        in_specs=[
            pl.BlockSpec(memory_space=pl.MemorySpace.ANY),
            pl.BlockSpec(memory_space=pl.MemorySpace.ANY),
        ],
        out_specs=pl.BlockSpec(memory_space=pl.MemorySpace.ANY),
        scratch_shapes=(
            [pltpu.VMEM((SLOTS * BATCH, DIM), jnp.float32)
             for _ in range(NBUF)]
            + [pltpu.SemaphoreType.DMA for _ in range(2 * NBUF)]
        ),
    )
    out = pl.pallas_call(
        _body,
        grid_spec=grid_spec,
        out_shape=jax.ShapeDtypeStruct((WINDOW * BATCH, DIM), jnp.float32),
    )(ptr, x, flat)
    return out.reshape(WINDOW, BATCH, DIM)


# speedup vs baseline: 1.0028x; 1.0028x over previous
import jax
import jax.numpy as jnp
from jax.experimental import pallas as pl
from jax.experimental.pallas import tpu as pltpu

WINDOW = 50
BATCH = 4096
DIM = 128
SLOTS = 2
NCHUNK = WINDOW // SLOTS
NBUF = 14


def _body(ptr_ref, x_ref, data_ref, out_ref, *scratch):
    bufs = scratch[:NBUF]
    rsems = scratch[NBUF:2 * NBUF]
    wsems = scratch[2 * NBUF:3 * NBUF]
    p = ptr_ref[0]

    def start_reads(c, buf, sem):
        for s in range(SLOTS):
            slot = c * SLOTS + s

            @pl.when(slot == p)
            def _from_x():
                pltpu.make_async_copy(
                    x_ref, buf.at[pl.ds(s * BATCH, BATCH)], sem).start()

            @pl.when(slot != p)
            def _from_data():
                pltpu.make_async_copy(
                    data_ref.at[pl.ds(slot * BATCH, BATCH)],
                    buf.at[pl.ds(s * BATCH, BATCH)], sem).start()

    def wait_reads(buf, sem):
        for s in range(SLOTS):
            pltpu.make_async_copy(
                data_ref.at[pl.ds(0, BATCH)],
                buf.at[pl.ds(s * BATCH, BATCH)], sem).wait()

    def wait_write(buf, sem):
        pltpu.make_async_copy(buf, out_ref.at[pl.ds(0, SLOTS * BATCH)],
                              sem).wait()

    for c in range(NBUF - 1):
        start_reads(c, bufs[c % NBUF], rsems[c % NBUF])
    for c in range(NCHUNK):
        b = c % NBUF
        wait_reads(bufs[b], rsems[b])
        pltpu.make_async_copy(
            bufs[b], out_ref.at[pl.ds(c * SLOTS * BATCH, SLOTS * BATCH)],
            wsems[b]).start()
        nxt = c + NBUF - 1
        if nxt < NCHUNK:
            bn = nxt % NBUF
            if nxt >= NBUF:
                wait_write(bufs[bn], wsems[bn])
            start_reads(nxt, bufs[bn], rsems[bn])
    for c in range(max(0, NCHUNK - NBUF), NCHUNK):
        b = c % NBUF
        wait_write(bufs[b], wsems[b])


def kernel(x, data, pointer):
    ptr = jnp.atleast_1d(jnp.asarray(pointer, dtype=jnp.int32))
    flat = data.reshape(WINDOW * BATCH, DIM)
    grid_spec = pltpu.PrefetchScalarGridSpec(
        num_scalar_prefetch=1,
        grid=(1,),
        in_specs=[
            pl.BlockSpec(memory_space=pl.MemorySpace.ANY),
            pl.BlockSpec(memory_space=pl.MemorySpace.ANY),
        ],
        out_specs=pl.BlockSpec(memory_space=pl.MemorySpace.ANY),
        scratch_shapes=(
            [pltpu.VMEM((SLOTS * BATCH, DIM), jnp.float32)
             for _ in range(NBUF)]
            + [pltpu.SemaphoreType.DMA for _ in range(2 * NBUF)]
        ),
    )
    out = pl.pallas_call(
        _body,
        grid_spec=grid_spec,
        out_shape=jax.ShapeDtypeStruct((WINDOW * BATCH, DIM), jnp.float32),
    )(ptr, x, flat)
    return out.reshape(WINDOW, BATCH, DIM)
